# baseline (speedup 1.0000x reference)
import functools

import jax
import jax.numpy as jnp
from jax import lax
from jax.experimental import pallas as pl
from jax.experimental.pallas import tpu as pltpu
from jax.experimental.pallas import tpu_sc as plsc

_NC = 2
_NS = 16
_NW = _NC * _NS


def _pack_body(tt_ref, out_ref):
    a = tt_ref[...]
    a3 = a.reshape(64, 64, 2)
    out_ref[...] = a3.transpose(1, 2, 0).reshape(64, 128)


def _gather_body(x_hbm, table_hbm, out_hbm, idx_all, rows, *sems,
                 per_w, chunk, nbuf):
    gsems, wsems = sems[:nbuf], sems[nbuf:]
    wid = lax.axis_index("s") * _NC + lax.axis_index("c")
    base = wid * per_w
    n_chunks = per_w // chunk

    pltpu.sync_copy(x_hbm.at[wid], idx_all)

    def gather(i, b):
        return pltpu.make_async_copy(
            table_hbm.at[idx_all.at[i]], rows.at[b], gsems[b])

    def writeback(i, b):
        return pltpu.make_async_copy(
            rows.at[b],
            out_hbm.at[pl.ds(base + i * chunk, chunk), pl.ds(0, 64)],
            wsems[b])

    for b in range(nbuf):
        gather(b, b).start()

    @pl.loop(0, n_chunks, step=nbuf)
    def _(c0):
        for b in range(nbuf):
            i = c0 + b
            gather(i, b).wait()
            writeback(i, b).start()
        for b in range(nbuf):
            i = c0 + b
            writeback(i, b).wait()

            @pl.when(i + nbuf < n_chunks)
            def _():
                gather(i + nbuf, b).start()


def kernel(x, table):
    b, s = x.shape
    n, d = table.shape
    dp = 128
    total = b * s
    per_w = total // _NW
    chunk = 256
    nbuf = 2
    n_chunks = per_w // chunk
    x_split = x.reshape(_NW, n_chunks, chunk).astype(jnp.int32)

    nblk = (n + 127) // 128
    pack = pl.pallas_call(
        _pack_body,
        grid=(nblk,),
        in_specs=[pl.BlockSpec((d, 128), lambda i: (0, i))],
        out_specs=pl.BlockSpec((64, 128), lambda i: (i, 0)),
        out_shape=jax.ShapeDtypeStruct((nblk * 64, 128), jnp.float32),
    )
    tpacked = pack(table.T).reshape(nblk * 128, d)

    mesh = plsc.VectorSubcoreMesh(core_axis_name="c", subcore_axis_name="s")
    gatherk = pl.kernel(
        functools.partial(_gather_body, per_w=per_w, chunk=chunk, nbuf=nbuf),
        out_type=jax.ShapeDtypeStruct((total, dp), jnp.float32),
        mesh=mesh,
        compiler_params=pltpu.CompilerParams(use_tc_tiling_on_sc=False),
        scratch_types=(
            [pltpu.VMEM((n_chunks, chunk), jnp.int32),
             pltpu.VMEM((nbuf, chunk, d), jnp.float32)]
            + [pltpu.SemaphoreType.DMA] * (2 * nbuf)
        ),
    )
    out = gatherk(x_split, tpacked)
    return out.reshape(b, s, dp)[:, :, :d]


# speedup vs baseline: 2.2112x; 2.2112x over previous
import functools

import jax
import jax.numpy as jnp
from jax import lax
from jax.experimental import pallas as pl
from jax.experimental.pallas import tpu as pltpu
from jax.experimental.pallas import tpu_sc as plsc

_NC = 2
_NS = 16
_NW = _NC * _NS


def _pack_body(tt_ref, out_ref, *, nrows):
    blk = pl.program_id(0)
    col = lax.broadcasted_iota(jnp.int32, (64, 128), 1) + 128 * blk
    a = jnp.where(col < nrows, tt_ref[...], 0.0)
    i0 = lax.broadcasted_iota(jnp.int32, (128, 64), 0)
    i1 = lax.broadcasted_iota(jnp.int32, (128, 64), 1)
    dims = (((0,), (1,)), ((), ()))
    for h in range(2):
        sel = (i0 == 2 * i1 + h).astype(jnp.float32)
        out_ref[:, 64 * h:64 * (h + 1)] = lax.dot_general(
            sel, a, dims, preferred_element_type=jnp.float32)


def _gather_body(x_hbm, table_hbm, out_hbm, idx_all, rows, *sems,
                 per_w, chunk, nbuf):
    gsems, wsems = sems[:nbuf], sems[nbuf:]
    wid = lax.axis_index("s") * _NC + lax.axis_index("c")
    base = wid * per_w
    n_chunks = per_w // chunk

    pltpu.sync_copy(x_hbm.at[wid], idx_all)

    def gather(i, b):
        return pltpu.make_async_copy(
            table_hbm.at[idx_all.at[i]], rows.at[b], gsems[b])

    def writeback(i, b):
        return pltpu.make_async_copy(
            rows.at[b],
            out_hbm.at[pl.ds(base + i * chunk, chunk), pl.ds(0, 64)],
            wsems[b])

    for b in range(nbuf):
        gather(b, b).start()

    @pl.loop(0, n_chunks, step=nbuf)
    def _(c0):
        for b in range(nbuf):
            i = c0 + b
            gather(i, b).wait()
            writeback(i, b).start()
        for b in range(nbuf):
            i = c0 + b
            writeback(i, b).wait()

            @pl.when(i + nbuf < n_chunks)
            def _():
                gather(i + nbuf, b).start()


def kernel(x, table):
    b, s = x.shape
    n, d = table.shape
    dp = 128
    total = b * s
    per_w = total // _NW
    chunk = 256
    nbuf = 2
    n_chunks = per_w // chunk
    x_split = x.reshape(_NW, n_chunks, chunk).astype(jnp.int32)

    nblk = (n + 127) // 128
    pack = pl.pallas_call(
        functools.partial(_pack_body, nrows=n),
        grid=(nblk,),
        in_specs=[pl.BlockSpec((d, 128), lambda i: (0, i))],
        out_specs=pl.BlockSpec((64, 128), lambda i: (i, 0)),
        out_shape=jax.ShapeDtypeStruct((nblk * 64, 128), jnp.float32),
    )
    tpacked = pack(table.T).reshape(nblk * 128, d)

    mesh = plsc.VectorSubcoreMesh(core_axis_name="c", subcore_axis_name="s")
    gatherk = pl.kernel(
        functools.partial(_gather_body, per_w=per_w, chunk=chunk, nbuf=nbuf),
        out_type=jax.ShapeDtypeStruct((total, dp), jnp.float32),
        mesh=mesh,
        compiler_params=pltpu.CompilerParams(use_tc_tiling_on_sc=False),
        scratch_types=(
            [pltpu.VMEM((n_chunks, chunk), jnp.int32),
             pltpu.VMEM((nbuf, chunk, d), jnp.float32)]
            + [pltpu.SemaphoreType.DMA] * (2 * nbuf)
        ),
    )
    out = gatherk(x_split, tpacked)
    return out.reshape(b, s, dp)[:, :, :d]


# speedup vs baseline: 2.3231x; 1.0506x over previous
import functools

import jax
import jax.numpy as jnp
from jax import lax
from jax.experimental import pallas as pl
from jax.experimental.pallas import tpu as pltpu
from jax.experimental.pallas import tpu_sc as plsc

_NC = 2
_NS = 16
_NW = _NC * _NS


def _pack_body(tt_ref, out_ref, *, nrows):
    del nrows
    t = tt_ref[...].T
    t3 = t.reshape(64, 2, 64)
    out_ref[:, 0:64] = t3[:, 0, :]
    out_ref[:, 64:128] = t3[:, 1, :]


def _gather_body(x_hbm, table_hbm, out_hbm, idx_all, rows, *sems,
                 per_w, chunk, nbuf):
    gsems, wsems = sems[:nbuf], sems[nbuf:]
    wid = lax.axis_index("s") * _NC + lax.axis_index("c")
    base = wid * per_w
    n_chunks = per_w // chunk

    pltpu.sync_copy(x_hbm.at[wid], idx_all)

    def gather(i, b):
        return pltpu.make_async_copy(
            table_hbm.at[idx_all.at[i]], rows.at[b], gsems[b])

    def writeback(i, b):
        return pltpu.make_async_copy(
            rows.at[b],
            out_hbm.at[pl.ds(base + i * chunk, chunk), pl.ds(0, 64)],
            wsems[b])

    for b in range(nbuf):
        gather(b, b).start()

    @pl.loop(0, n_chunks, step=nbuf)
    def _(c0):
        for b in range(nbuf):
            i = c0 + b
            gather(i, b).wait()
            writeback(i, b).start()
        for b in range(nbuf):
            i = c0 + b
            writeback(i, b).wait()

            @pl.when(i + nbuf < n_chunks)
            def _():
                gather(i + nbuf, b).start()


def kernel(x, table):
    b, s = x.shape
    n, d = table.shape
    dp = 128
    total = b * s
    per_w = total // _NW
    chunk = 256
    nbuf = 2
    n_chunks = per_w // chunk
    x_split = x.reshape(_NW, n_chunks, chunk).astype(jnp.int32)

    nblk = (n + 127) // 128
    pack = pl.pallas_call(
        functools.partial(_pack_body, nrows=n),
        grid=(nblk,),
        in_specs=[pl.BlockSpec((d, 128), lambda i: (0, i))],
        out_specs=pl.BlockSpec((64, 128), lambda i: (i, 0)),
        out_shape=jax.ShapeDtypeStruct((nblk * 64, 128), jnp.float32),
    )
    tpacked = pack(table.T).reshape(nblk * 128, d)

    mesh = plsc.VectorSubcoreMesh(core_axis_name="c", subcore_axis_name="s")
    gatherk = pl.kernel(
        functools.partial(_gather_body, per_w=per_w, chunk=chunk, nbuf=nbuf),
        out_type=jax.ShapeDtypeStruct((total, dp), jnp.float32),
        mesh=mesh,
        compiler_params=pltpu.CompilerParams(use_tc_tiling_on_sc=False),
        scratch_types=(
            [pltpu.VMEM((n_chunks, chunk), jnp.int32),
             pltpu.VMEM((nbuf, chunk, d), jnp.float32)]
            + [pltpu.SemaphoreType.DMA] * (2 * nbuf)
        ),
    )
    out = gatherk(x_split, tpacked)
    return out.reshape(b, s, dp)[:, :, :d]


# speedup vs baseline: 13.8763x; 5.9731x over previous
import functools

import jax
import jax.numpy as jnp
from jax import lax
from jax.experimental import pallas as pl
from jax.experimental.pallas import tpu as pltpu
from jax.experimental.pallas import tpu_sc as plsc

_NC = 2
_NS = 16
_NW = _NC * _NS


def _pack_body(tt_ref, out_ref, *, nsub):
    for sub in range(nsub):
        t = tt_ref[:, pl.ds(128 * sub, 128)].T
        t3 = t.reshape(64, 2, 64)
        out_ref[pl.ds(64 * sub, 64), 0:64] = t3[:, 0, :]
        out_ref[pl.ds(64 * sub, 64), 64:128] = t3[:, 1, :]


def _gather_body(x_hbm, table_hbm, out_hbm, idx_all, rows, *sems,
                 per_w, chunk, nbuf):
    gsems, wsems = sems[:nbuf], sems[nbuf:]
    wid = lax.axis_index("s") * _NC + lax.axis_index("c")
    base = wid * per_w
    n_chunks = per_w // chunk

    pltpu.sync_copy(x_hbm.at[wid], idx_all)

    def gather(i, b):
        return pltpu.make_async_copy(
            table_hbm.at[idx_all.at[i]], rows.at[b], gsems[b])

    def writeback(i, b):
        return pltpu.make_async_copy(
            rows.at[b],
            out_hbm.at[pl.ds(base + i * chunk, chunk), pl.ds(0, 64)],
            wsems[b])

    for b in range(nbuf):
        gather(b, b).start()

    @pl.loop(0, n_chunks, step=nbuf)
    def _(c0):
        for b in range(nbuf):
            i = c0 + b
            gather(i, b).wait()
            writeback(i, b).start()
        for b in range(nbuf):
            i = c0 + b
            writeback(i, b).wait()

            @pl.when(i + nbuf < n_chunks)
            def _():
                gather(i + nbuf, b).start()


def kernel(x, table):
    b, s = x.shape
    n, d = table.shape
    dp = 128
    total = b * s
    per_w = total // _NW
    chunk = 256
    nbuf = 2
    n_chunks = per_w // chunk
    x_split = x.reshape(_NW, n_chunks, chunk).astype(jnp.int32)

    nsub = 32
    cw = 128 * nsub
    nstep = (n + cw - 1) // cw
    pack = pl.pallas_call(
        functools.partial(_pack_body, nsub=nsub),
        grid=(nstep,),
        in_specs=[pl.BlockSpec((d, cw), lambda i: (0, i))],
        out_specs=pl.BlockSpec((64 * nsub, 128), lambda i: (i, 0)),
        out_shape=jax.ShapeDtypeStruct((nstep * 64 * nsub, 128), jnp.float32),
    )
    tpacked = pack(table.T).reshape(nstep * cw, d)

    mesh = plsc.VectorSubcoreMesh(core_axis_name="c", subcore_axis_name="s")
    gatherk = pl.kernel(
        functools.partial(_gather_body, per_w=per_w, chunk=chunk, nbuf=nbuf),
        out_type=jax.ShapeDtypeStruct((total, dp), jnp.float32),
        mesh=mesh,
        compiler_params=pltpu.CompilerParams(use_tc_tiling_on_sc=False),
        scratch_types=(
            [pltpu.VMEM((n_chunks, chunk), jnp.int32),
             pltpu.VMEM((nbuf, chunk, d), jnp.float32)]
            + [pltpu.SemaphoreType.DMA] * (2 * nbuf)
        ),
    )
    out = gatherk(x_split, tpacked)
    return out.reshape(b, s, dp)[:, :, :d]


# speedup vs baseline: 14.4210x; 1.0393x over previous
import functools

import jax
import jax.numpy as jnp
from jax import lax
from jax.experimental import pallas as pl
from jax.experimental.pallas import tpu as pltpu
from jax.experimental.pallas import tpu_sc as plsc

_NC = 2
_NS = 16
_NW = _NC * _NS


def _pack_body(tt_ref, out_ref, *, nsub):
    for sub in range(nsub):
        t = tt_ref[:, pl.ds(128 * sub, 128)].T
        t3 = t.reshape(64, 2, 64)
        out_ref[pl.ds(64 * sub, 64), 0:64] = t3[:, 0, :]
        out_ref[pl.ds(64 * sub, 64), 64:128] = t3[:, 1, :]


def _gather_body(x_hbm, table_hbm, out_hbm, idx_all, rows, *sems,
                 per_w, chunk, nbuf):
    gsems, wsems = sems[:nbuf], sems[nbuf:]
    wid = lax.axis_index("s") * _NC + lax.axis_index("c")
    base = wid * per_w
    n_chunks = per_w // chunk

    pltpu.sync_copy(x_hbm.at[wid], idx_all)

    def gather(i, b):
        return pltpu.make_async_copy(
            table_hbm.at[idx_all.at[i]], rows.at[b], gsems[b])

    def writeback(i, b):
        return pltpu.make_async_copy(
            rows.at[b],
            out_hbm.at[pl.ds(base + i * chunk, chunk), pl.ds(0, 64)],
            wsems[b])

    for b in range(nbuf):
        gather(b, b).start()

    @pl.loop(0, n_chunks, step=nbuf)
    def _(c0):
        for b in range(nbuf):
            i = c0 + b
            gather(i, b).wait()
            writeback(i, b).start()
        for b in range(nbuf):
            i = c0 + b
            writeback(i, b).wait()

            @pl.when(i + nbuf < n_chunks)
            def _():
                gather(i + nbuf, b).start()


def kernel(x, table):
    b, s = x.shape
    n, d = table.shape
    dp = 128
    total = b * s
    per_w = total // _NW
    chunk = 256
    nbuf = 2
    n_chunks = per_w // chunk
    x_split = x.reshape(_NW, n_chunks, chunk).astype(jnp.int32)

    nsub = 64
    cw = 128 * nsub
    nstep = (n + cw - 1) // cw
    pack = pl.pallas_call(
        functools.partial(_pack_body, nsub=nsub),
        grid=(nstep,),
        in_specs=[pl.BlockSpec((d, cw), lambda i: (0, i))],
        out_specs=pl.BlockSpec((64 * nsub, 128), lambda i: (i, 0)),
        out_shape=jax.ShapeDtypeStruct((nstep * 64 * nsub, 128), jnp.float32),
    )
    tpacked = pack(table.T).reshape(nstep * cw, d)

    mesh = plsc.VectorSubcoreMesh(core_axis_name="c", subcore_axis_name="s")
    gatherk = pl.kernel(
        functools.partial(_gather_body, per_w=per_w, chunk=chunk, nbuf=nbuf),
        out_type=jax.ShapeDtypeStruct((total, dp), jnp.float32),
        mesh=mesh,
        compiler_params=pltpu.CompilerParams(use_tc_tiling_on_sc=False),
        scratch_types=(
            [pltpu.VMEM((n_chunks, chunk), jnp.int32),
             pltpu.VMEM((nbuf, chunk, d), jnp.float32)]
            + [pltpu.SemaphoreType.DMA] * (2 * nbuf)
        ),
    )
    out = gatherk(x_split, tpacked)
    return out.reshape(b, s, dp)[:, :, :d]


# speedup vs baseline: 14.5030x; 1.0057x over previous
import functools

import jax
import jax.numpy as jnp
from jax import lax
from jax.experimental import pallas as pl
from jax.experimental.pallas import tpu as pltpu
from jax.experimental.pallas import tpu_sc as plsc

_NC = 2
_NS = 16
_NW = _NC * _NS


def _pack_body(tt_ref, out_ref, *, nsub):
    for sub in range(nsub):
        t = tt_ref[:, pl.ds(128 * sub, 128)].T
        t3 = t.reshape(64, 2, 64)
        out_ref[pl.ds(64 * sub, 64), 0:64] = t3[:, 0, :]
        out_ref[pl.ds(64 * sub, 64), 64:128] = t3[:, 1, :]


def _gather_body(x_hbm, table_hbm, out_hbm, idx_all, rows, *sems,
                 per_w, chunk, nbuf):
    gsems, wsems = sems[:nbuf], sems[nbuf:]
    wid = lax.axis_index("s") * _NC + lax.axis_index("c")
    base = wid * per_w
    n_chunks = per_w // chunk

    pltpu.sync_copy(x_hbm.at[wid], idx_all)

    def gather(i, b):
        return pltpu.make_async_copy(
            table_hbm.at[idx_all.at[i]], rows.at[b], gsems[b])

    def writeback(i, b):
        return pltpu.make_async_copy(
            rows.at[b],
            out_hbm.at[pl.ds(base + i * chunk, chunk), pl.ds(0, 64)],
            wsems[b])

    for b in range(nbuf):
        gather(b, b).start()

    @pl.loop(0, n_chunks, step=nbuf)
    def _(c0):
        for b in range(nbuf):
            i = c0 + b
            gather(i, b).wait()
            writeback(i, b).start()
        for b in range(nbuf):
            i = c0 + b
            writeback(i, b).wait()

            @pl.when(i + nbuf < n_chunks)
            def _():
                gather(i + nbuf, b).start()


def kernel(x, table):
    b, s = x.shape
    n, d = table.shape
    dp = 128
    total = b * s
    per_w = total // _NW
    chunk = 256
    nbuf = 2
    n_chunks = per_w // chunk
    x_split = x.reshape(_NW, n_chunks, chunk).astype(jnp.int32)

    nsub = 128
    cw = 128 * nsub
    nstep = (n + cw - 1) // cw
    pack = pl.pallas_call(
        functools.partial(_pack_body, nsub=nsub),
        grid=(nstep,),
        in_specs=[pl.BlockSpec((d, cw), lambda i: (0, i))],
        out_specs=pl.BlockSpec((64 * nsub, 128), lambda i: (i, 0)),
        out_shape=jax.ShapeDtypeStruct((nstep * 64 * nsub, 128), jnp.float32),
    )
    tpacked = pack(table.T).reshape(nstep * cw, d)

    mesh = plsc.VectorSubcoreMesh(core_axis_name="c", subcore_axis_name="s")
    gatherk = pl.kernel(
        functools.partial(_gather_body, per_w=per_w, chunk=chunk, nbuf=nbuf),
        out_type=jax.ShapeDtypeStruct((total, dp), jnp.float32),
        mesh=mesh,
        compiler_params=pltpu.CompilerParams(use_tc_tiling_on_sc=False),
        scratch_types=(
            [pltpu.VMEM((n_chunks, chunk), jnp.int32),
             pltpu.VMEM((nbuf, chunk, d), jnp.float32)]
            + [pltpu.SemaphoreType.DMA] * (2 * nbuf)
        ),
    )
    out = gatherk(x_split, tpacked)
    return out.reshape(b, s, dp)[:, :, :d]


# speedup vs baseline: 14.7188x; 1.0149x over previous
import functools

import jax
import jax.numpy as jnp
from jax import lax
from jax.experimental import pallas as pl
from jax.experimental.pallas import tpu as pltpu
from jax.experimental.pallas import tpu_sc as plsc

_NC = 2
_NS = 16
_NW = _NC * _NS


def _pack_body(tt_ref, out_ref, *, nsub):
    for sub in range(nsub):
        t = tt_ref[:, pl.ds(128 * sub, 128)].T
        t3 = t.reshape(64, 2, 64)
        out_ref[pl.ds(64 * sub, 64), 0:64] = t3[:, 0, :]
        out_ref[pl.ds(64 * sub, 64), 64:128] = t3[:, 1, :]


def _gather_body(x_hbm, table_hbm, out_hbm, idx_all, rows, *sems,
                 per_w, chunk, nbuf):
    gsems, wsems = sems[:nbuf], sems[nbuf:]
    wid = lax.axis_index("s") * _NC + lax.axis_index("c")
    base = wid * per_w
    n_chunks = per_w // chunk

    pltpu.sync_copy(x_hbm.at[wid], idx_all)

    def gather(i, b):
        return pltpu.make_async_copy(
            table_hbm.at[idx_all.at[i]], rows.at[b], gsems[b])

    def writeback(i, b):
        return pltpu.make_async_copy(
            rows.at[b],
            out_hbm.at[pl.ds(base + i * chunk, chunk), pl.ds(0, 64)],
            wsems[b])

    for b in range(nbuf):
        gather(b, b).start()

    @pl.loop(0, n_chunks, step=nbuf)
    def _(c0):
        for b in range(nbuf):
            i = c0 + b
            gather(i, b).wait()
            writeback(i, b).start()
        for b in range(nbuf):
            i = c0 + b
            writeback(i, b).wait()

            @pl.when(i + nbuf < n_chunks)
            def _():
                gather(i + nbuf, b).start()


def kernel(x, table):
    b, s = x.shape
    n, d = table.shape
    dp = 128
    total = b * s
    per_w = total // _NW
    chunk = 512
    nbuf = 2
    n_chunks = per_w // chunk
    x_split = x.reshape(_NW, n_chunks, chunk).astype(jnp.int32)

    nsub = 128
    cw = 128 * nsub
    nstep = (n + cw - 1) // cw
    pack = pl.pallas_call(
        functools.partial(_pack_body, nsub=nsub),
        grid=(nstep,),
        in_specs=[pl.BlockSpec((d, cw), lambda i: (0, i))],
        out_specs=pl.BlockSpec((64 * nsub, 128), lambda i: (i, 0)),
        out_shape=jax.ShapeDtypeStruct((nstep * 64 * nsub, 128), jnp.float32),
    )
    tpacked = pack(table.T).reshape(nstep * cw, d)

    mesh = plsc.VectorSubcoreMesh(core_axis_name="c", subcore_axis_name="s")
    gatherk = pl.kernel(
        functools.partial(_gather_body, per_w=per_w, chunk=chunk, nbuf=nbuf),
        out_type=jax.ShapeDtypeStruct((total, dp), jnp.float32),
        mesh=mesh,
        compiler_params=pltpu.CompilerParams(use_tc_tiling_on_sc=False),
        scratch_types=(
            [pltpu.VMEM((n_chunks, chunk), jnp.int32),
             pltpu.VMEM((nbuf, chunk, d), jnp.float32)]
            + [pltpu.SemaphoreType.DMA] * (2 * nbuf)
        ),
    )
    out = gatherk(x_split, tpacked)
    return out.reshape(b, s, dp)[:, :, :d]
